# baseline (speedup 1.0000x reference)
import functools

import jax
import jax.numpy as jnp
from jax import lax
from jax.experimental import pallas as pl
from jax.experimental.pallas import tpu as pltpu

B = 4096
C = 8
D = 1024
E = 8
H = 1024
K = 128
CAP_F = 2.4

GB = 512
NBG = B // GB
XB = 512
NBX = B // XB
HB = 128
NH = H // HB

_CAPACITY = CAP_F * B / E


def _top2(g, idx, sentinel):
    v1 = jnp.max(g, axis=1, keepdims=True)
    i1 = jnp.min(jnp.where(g == v1, idx, sentinel), axis=1, keepdims=True)
    g2 = jnp.where(idx == i1, -jnp.inf, g)
    v2 = jnp.max(g2, axis=1, keepdims=True)
    i2 = jnp.min(jnp.where(g2 == v2, idx, sentinel), axis=1, keepdims=True)
    return v1, i1, v2, i2


def _gate_routing_body(x_ref, wg_ref, bg_ref, out_ref, counts_ref):
    b = pl.program_id(0)

    gate = jnp.zeros((GB, E), jnp.float32)
    for c in range(C):
        logits = jnp.dot(x_ref[:, c, :], wg_ref[...],
                         preferred_element_type=jnp.float32) + bg_ref[...]
        gate = gate + jax.nn.softmax(logits, axis=-1)
    gate = gate / C

    idx = lax.broadcasted_iota(jnp.int32, (GB, E), 1)
    v1, i1, v2, i2 = _top2(gate, idx, E)
    tophot = ((idx == i1) | (idx == i2)).astype(jnp.float32)

    ri = lax.broadcasted_iota(jnp.int32, (GB, GB), 0)
    ci = lax.broadcasted_iota(jnp.int32, (GB, GB), 1)
    tri = (ri >= ci).astype(jnp.float32)
    csum = jnp.dot(tri, tophot, preferred_element_type=jnp.float32)

    @pl.when(b == 0)
    def _():
        counts_ref[...] = jnp.zeros((1, E), jnp.float32)

    carry = counts_ref[...]
    total = csum + carry
    counts_ref[...] = total[GB - 1:GB, :]
    mask = total > _CAPACITY
    gm = jnp.where(mask, 0.0, gate)

    w1, j1, w2, j2 = _top2(gm, idx, E)
    out_ref[...] = (jnp.where(idx == j1, w1, 0.0)
                    + jnp.where(idx == j2, w2, 0.0))


def _experts_body(x_ref, w1_ref, b1_ref, w2_ref, b2_ref, gate_ref,
                  out_ref, diag_ref, ex_ref):
    e = pl.program_id(1)
    nh = pl.program_id(2)

    acc = jnp.zeros((XB, HB), jnp.float32)
    for c in range(C):
        acc = acc + jnp.dot(x_ref[:, c, :], w1_ref[0, c * D:(c + 1) * D, :],
                            preferred_element_type=jnp.float32)
    h = jnp.maximum(acc + b1_ref[0, 0], 0.0)
    part = jnp.dot(h, w2_ref[0], preferred_element_type=jnp.float32)

    @pl.when(nh == 0)
    def _():
        ex_ref[e] = part

    @pl.when(nh != 0)
    def _():
        ex_ref[e] = ex_ref[e] + part

    @pl.when(nh == NH - 1)
    def _():
        ex_ref[e] = ex_ref[e] + b2_ref[0]

    @pl.when((e == E - 1) & (nh == NH - 1))
    def _():
        ex = ex_ref[...]
        mean = jnp.mean(ex, axis=0)
        cent = ex - mean[None]
        diag_ref[...] = jnp.sum(cent * cent, axis=0) / (E - 1)
        g = gate_ref[...]
        acc = jnp.zeros((XB, K), jnp.float32)
        for ee in range(E):
            acc = acc + g[:, ee:ee + 1] * ex[ee]
        out_ref[...] = acc


@jax.jit
def kernel(x, Wg, bg, W1, b1, W2, b2):
    gate_final = pl.pallas_call(
        _gate_routing_body,
        grid=(NBG,),
        in_specs=[
            pl.BlockSpec((GB, C, D), lambda b: (b, 0, 0)),
            pl.BlockSpec((D, E), lambda b: (0, 0)),
            pl.BlockSpec((1, E), lambda b: (0, 0)),
        ],
        out_specs=pl.BlockSpec((GB, E), lambda b: (b, 0)),
        out_shape=jax.ShapeDtypeStruct((B, E), jnp.float32),
        scratch_shapes=[pltpu.VMEM((1, E), jnp.float32)],
    )(x, Wg, bg.reshape(1, E))

    out, diag = pl.pallas_call(
        _experts_body,
        grid=(NBX, E, NH),
        in_specs=[
            pl.BlockSpec((XB, C, D), lambda b, e, nh: (b, 0, 0)),
            pl.BlockSpec((1, C * D, HB), lambda b, e, nh: (e, 0, nh)),
            pl.BlockSpec((1, 1, 1, HB), lambda b, e, nh: (e, nh, 0, 0)),
            pl.BlockSpec((1, HB, K), lambda b, e, nh: (e, nh, 0)),
            pl.BlockSpec((1, 1, K), lambda b, e, nh: (e, 0, 0)),
            pl.BlockSpec((XB, E), lambda b, e, nh: (b, 0)),
        ],
        out_specs=[
            pl.BlockSpec((XB, K), lambda b, e, nh: (b, 0)),
            pl.BlockSpec((XB, K), lambda b, e, nh: (b, 0)),
        ],
        out_shape=[
            jax.ShapeDtypeStruct((B, K), jnp.float32),
            jax.ShapeDtypeStruct((B, K), jnp.float32),
        ],
        scratch_shapes=[pltpu.VMEM((E, XB, K), jnp.float32)],
    )(x, W1, b1.reshape(E, NH, 1, HB), W2, b2.reshape(E, 1, K), gate_final)

    return (out, diag)


# speedup vs baseline: 3.9496x; 3.9496x over previous
import functools

import jax
import jax.numpy as jnp
from jax import lax
from jax.experimental import pallas as pl
from jax.experimental.pallas import tpu as pltpu

B = 4096
C = 8
D = 1024
E = 8
H = 1024
K = 128
CAP_F = 2.4

GB = 512
NBG = B // GB
XB = 1024
NBX = B // XB
HB = 256
NH = H // HB

_CAPACITY = CAP_F * B / E


def _top2(g, idx, sentinel):
    v1 = jnp.max(g, axis=1, keepdims=True)
    i1 = jnp.min(jnp.where(g == v1, idx, sentinel), axis=1, keepdims=True)
    g2 = jnp.where(idx == i1, -jnp.inf, g)
    v2 = jnp.max(g2, axis=1, keepdims=True)
    i2 = jnp.min(jnp.where(g2 == v2, idx, sentinel), axis=1, keepdims=True)
    return v1, i1, v2, i2


def _gate_routing_body(x_ref, wg_ref, bg_ref, out_ref, counts_ref):
    b = pl.program_id(0)

    gate = jnp.zeros((GB, E), jnp.float32)
    for c in range(C):
        logits = jnp.dot(x_ref[:, c, :], wg_ref[...],
                         preferred_element_type=jnp.float32) + bg_ref[...]
        gate = gate + jax.nn.softmax(logits, axis=-1)
    gate = gate / C

    idx = lax.broadcasted_iota(jnp.int32, (GB, E), 1)
    v1, i1, v2, i2 = _top2(gate, idx, E)
    tophot = ((idx == i1) | (idx == i2)).astype(jnp.float32)

    ri = lax.broadcasted_iota(jnp.int32, (GB, GB), 0)
    ci = lax.broadcasted_iota(jnp.int32, (GB, GB), 1)
    tri = (ri >= ci).astype(jnp.float32)
    csum = jnp.dot(tri, tophot, preferred_element_type=jnp.float32)

    @pl.when(b == 0)
    def _():
        counts_ref[...] = jnp.zeros((1, E), jnp.float32)

    carry = counts_ref[...]
    total = csum + carry
    counts_ref[...] = total[GB - 1:GB, :]
    mask = total > _CAPACITY
    gm = jnp.where(mask, 0.0, gate)

    w1, j1, w2, j2 = _top2(gm, idx, E)
    out_ref[...] = (jnp.where(idx == j1, w1, 0.0)
                    + jnp.where(idx == j2, w2, 0.0))


def _experts_body(x_ref, w1_ref, b1_ref, w2_ref, b2_ref, gate_ref,
                  out_ref, diag_ref, ex_ref):
    e = pl.program_id(1)
    nh = pl.program_id(2)

    h = jnp.maximum(
        jnp.dot(x_ref[...], w1_ref[0], preferred_element_type=jnp.float32)
        + b1_ref[0, 0], 0.0)
    part = jnp.dot(h, w2_ref[0], preferred_element_type=jnp.float32)

    @pl.when(nh == 0)
    def _():
        ex_ref[e] = part

    @pl.when(nh != 0)
    def _():
        ex_ref[e] = ex_ref[e] + part

    @pl.when(nh == NH - 1)
    def _():
        ex_ref[e] = ex_ref[e] + b2_ref[0]

    @pl.when((e == E - 1) & (nh == NH - 1))
    def _():
        ex = ex_ref[...]
        mean = jnp.mean(ex, axis=0)
        cent = ex - mean[None]
        diag_ref[...] = jnp.sum(cent * cent, axis=0) / (E - 1)
        g = gate_ref[...]
        acc = jnp.zeros((XB, K), jnp.float32)
        for ee in range(E):
            acc = acc + g[:, ee:ee + 1] * ex[ee]
        out_ref[...] = acc


@jax.jit
def kernel(x, Wg, bg, W1, b1, W2, b2):
    gate_final = pl.pallas_call(
        _gate_routing_body,
        grid=(NBG,),
        in_specs=[
            pl.BlockSpec((GB, C, D), lambda b: (b, 0, 0)),
            pl.BlockSpec((D, E), lambda b: (0, 0)),
            pl.BlockSpec((1, E), lambda b: (0, 0)),
        ],
        out_specs=pl.BlockSpec((GB, E), lambda b: (b, 0)),
        out_shape=jax.ShapeDtypeStruct((B, E), jnp.float32),
        scratch_shapes=[pltpu.VMEM((1, E), jnp.float32)],
    )(x, Wg, bg.reshape(1, E))

    xf = x.reshape(B, C * D).astype(jnp.bfloat16)
    out, diag = pl.pallas_call(
        _experts_body,
        grid=(NBX, E, NH),
        in_specs=[
            pl.BlockSpec((XB, C * D), lambda b, e, nh: (b, 0)),
            pl.BlockSpec((1, C * D, HB), lambda b, e, nh: (e, 0, nh)),
            pl.BlockSpec((1, 1, 1, HB), lambda b, e, nh: (e, nh, 0, 0)),
            pl.BlockSpec((1, HB, K), lambda b, e, nh: (e, nh, 0)),
            pl.BlockSpec((1, 1, K), lambda b, e, nh: (e, 0, 0)),
            pl.BlockSpec((XB, E), lambda b, e, nh: (b, 0)),
        ],
        out_specs=[
            pl.BlockSpec((XB, K), lambda b, e, nh: (b, 0)),
            pl.BlockSpec((XB, K), lambda b, e, nh: (b, 0)),
        ],
        out_shape=[
            jax.ShapeDtypeStruct((B, K), jnp.float32),
            jax.ShapeDtypeStruct((B, K), jnp.float32),
        ],
        scratch_shapes=[pltpu.VMEM((E, XB, K), jnp.float32)],
    )(xf, W1.astype(jnp.bfloat16), b1.reshape(E, NH, 1, HB),
      W2.astype(jnp.bfloat16), b2.reshape(E, 1, K), gate_final)

    return (out, diag)
